# Initial kernel scaffold; baseline (speedup 1.0000x reference)
import jax
import jax.numpy as jnp
from jax.experimental import pallas as pl


def kernel(meta_vec, x, edge_index, Ws1, bs1, Ws2, bs2, Wso, bso, Wg1, bg1, Wg2, bg2, Wg3, bg3, Wf1, bf1, Wf2, bf2, Wfo, bfo):
    raise NotImplementedError("write your pallas kernel here")


# speedup vs baseline: 61.1230x; 61.1230x over previous
import functools

import jax
import jax.numpy as jnp
from jax import lax
from jax.experimental import pallas as pl
from jax.experimental.pallas import tpu as pltpu
from jax.experimental.pallas import tpu_sc as plsc

N_NODES = 100000
N_EDGES = 3200000
NPAD = 100352
R, L = 784, 128
NC, NS = 2, 16
NW = NC * NS
CH = 80
K = 8
EPW = N_EDGES // NW
RPW = 1280
ROWS = 256
NBLK = 5
EPT = N_EDGES // NS
RPT = 2560
ROWS2 = 256
NBLK2 = 10
SLICE = NPAD // NS
NDUMMY = NPAD - N_NODES

_mesh = plsc.VectorSubcoreMesh(core_axis_name="c", subcore_axis_name="s")


def _fill(ref, n, value):
    val = jnp.full((16,), value, ref.dtype)

    def body(i, _):
        ref[pl.ds(i * 16, 16)] = val
        return 0

    lax.fori_loop(0, n // 16, body, 0)


@functools.partial(
    pl.kernel,
    out_type=jax.ShapeDtypeStruct((NC, NPAD), jnp.float32),
    mesh=_mesh,
    scratch_types=[
        pltpu.VMEM((ROWS, CH), jnp.int32),
        pltpu.VMEM((CH,), jnp.float32),
        pltpu.VMEM((SLICE,), jnp.float32),
        pltpu.VMEM_SHARED((NPAD,), jnp.float32),
        pltpu.SemaphoreType.DMA,
    ],
)
def _deg_kernel(dst_hbm, out_hbm, idx_v, ones_v, zer_v, acc_sh, ssem):
    c = lax.axis_index("c")
    s = lax.axis_index("s")
    wid = c * NS + s
    _fill(ones_v, CH, 1.0)
    _fill(zer_v, SLICE, 0.0)
    pltpu.sync_copy(zer_v, acc_sh.at[pl.ds(s * SLICE, SLICE)])
    plsc.subcore_barrier()

    for b in range(NBLK):
        pltpu.sync_copy(dst_hbm.at[wid, pl.ds(b * ROWS, ROWS)], idx_v)

        def step(jb, _):
            hs = [pltpu.async_copy(ones_v, acc_sh.at[idx_v.at[jb * K + i]],
                                   ssem, add=True) for i in range(K)]
            for h in hs:
                h.wait()
            return 0

        lax.fori_loop(0, ROWS // K, step, 0)

    plsc.subcore_barrier()
    pltpu.sync_copy(acc_sh.at[pl.ds(s * SLICE, SLICE)],
                    out_hbm.at[c, pl.ds(s * SLICE, SLICE)])


@functools.partial(
    pl.kernel,
    out_type=(jax.ShapeDtypeStruct((NC, NPAD), jnp.float32),
              jax.ShapeDtypeStruct((NC, NPAD), jnp.float32)),
    mesh=_mesh,
    scratch_types=[
        pltpu.VMEM((ROWS, CH), jnp.int32),
        pltpu.VMEM((ROWS, CH), jnp.int32),
        pltpu.VMEM((K, CH), jnp.float32),
        pltpu.VMEM((K, CH), jnp.float32),
        pltpu.VMEM((SLICE,), jnp.float32),
        pltpu.VMEM_SHARED((NPAD,), jnp.float32),
        pltpu.VMEM_SHARED((NPAD,), jnp.float32),
        pltpu.VMEM_SHARED((NPAD,), jnp.float32),
        pltpu.VMEM_SHARED((NPAD,), jnp.float32),
        pltpu.SemaphoreType.DMA,
        pltpu.SemaphoreType.DMA,
    ],
)
def _as_kernel(src_hbm, dst_hbm, u_hbm, v_hbm, a_out, s_out,
               sidx_v, didx_v, uva_v, vva_v, zer_v, u_sh, v_sh, a_sh, s_sh,
               gsem, ssem):
    c = lax.axis_index("c")
    s = lax.axis_index("s")
    wid = c * NS + s
    sl = pl.ds(s * SLICE, SLICE)
    _fill(zer_v, SLICE, 0.0)
    pltpu.sync_copy(zer_v, a_sh.at[sl])
    pltpu.sync_copy(zer_v, s_sh.at[sl])
    pltpu.sync_copy(u_hbm.at[sl], u_sh.at[sl])
    pltpu.sync_copy(v_hbm.at[sl], v_sh.at[sl])
    plsc.subcore_barrier()

    for b in range(NBLK):
        pltpu.sync_copy(src_hbm.at[wid, pl.ds(b * ROWS, ROWS)], sidx_v)
        pltpu.sync_copy(dst_hbm.at[wid, pl.ds(b * ROWS, ROWS)], didx_v)

        def step(jb, _):
            gs = []
            for i in range(K):
                j = jb * K + i
                gs.append(pltpu.async_copy(v_sh.at[sidx_v.at[j]],
                                           vva_v.at[i], gsem))
                gs.append(pltpu.async_copy(u_sh.at[didx_v.at[j]],
                                           uva_v.at[i], gsem))
            for h in gs:
                h.wait()
            ss = []
            for i in range(K):
                j = jb * K + i
                ss.append(pltpu.async_copy(vva_v.at[i],
                                           a_sh.at[didx_v.at[j]], ssem,
                                           add=True))
                ss.append(pltpu.async_copy(uva_v.at[i],
                                           s_sh.at[sidx_v.at[j]], ssem,
                                           add=True))
            for h in ss:
                h.wait()
            return 0

        lax.fori_loop(0, ROWS // K, step, 0)

    plsc.subcore_barrier()
    pltpu.sync_copy(a_sh.at[sl], a_out.at[c, sl])
    pltpu.sync_copy(s_sh.at[sl], s_out.at[c, sl])


@functools.partial(
    pl.kernel,
    out_type=jax.ShapeDtypeStruct((NC, NPAD, 8), jnp.float32),
    mesh=_mesh,
    compiler_params=pltpu.CompilerParams(use_tc_tiling_on_sc=False),
    scratch_types=[
        pltpu.VMEM((ROWS2, CH), jnp.int32),
        pltpu.VMEM((ROWS2, CH), jnp.int32),
        pltpu.VMEM((K, CH, 8), jnp.float32),
        pltpu.VMEM_SHARED((NPAD, 8), jnp.float32),
        pltpu.SemaphoreType.DMA,
        pltpu.SemaphoreType.DMA,
    ],
)
def _m_kernel(src_hbm, dst_hbm, qd_hbm, zer_hbm, out_hbm,
              sidx_v, didx_v, rows_v, m_sh, gsem, ssem):
    c = lax.axis_index("c")
    s = lax.axis_index("s")
    sl = pl.ds(s * SLICE, SLICE)
    for kk in range(8):
        pltpu.sync_copy(zer_hbm,
                        m_sh.at[pl.ds(s * SLICE + kk * (SLICE // 8),
                                      SLICE // 8)])
    plsc.subcore_barrier()

    for b in range(NBLK2):
        pltpu.sync_copy(src_hbm.at[c, s, pl.ds(b * ROWS2, ROWS2)], sidx_v)
        pltpu.sync_copy(dst_hbm.at[s, pl.ds(b * ROWS2, ROWS2)], didx_v)

        def step(jb, _):
            gs = [pltpu.async_copy(qd_hbm.at[sidx_v.at[jb * K + i]],
                                   rows_v.at[i], gsem) for i in range(K)]
            for h in gs:
                h.wait()
            ss = [pltpu.async_copy(rows_v.at[i],
                                   m_sh.at[didx_v.at[jb * K + i]], ssem,
                                   add=True) for i in range(K)]
            for h in ss:
                h.wait()
            return 0

        lax.fori_loop(0, ROWS2 // K, step, 0)

    plsc.subcore_barrier()
    pltpu.sync_copy(m_sh.at[sl], out_hbm.at[c, sl])


def _tc1_body(deg_ref, x_ref, dinv_ref, v_ref):
    deg = deg_ref[0] + deg_ref[1] + 1.0
    dinv = lax.rsqrt(deg)
    dinv_ref[...] = dinv
    v_ref[...] = dinv * x_ref[...]


def _tc2_body(a_ref, s_ref, dinv_ref, x_ref, mask_ref, wg1_ref, bg1_ref,
              wg2_ref, q_ref, qd_ref, w_ref):
    dinv = dinv_ref[...]
    dinv2 = dinv * dinv
    af = dinv * (a_ref[0] + a_ref[1]) + dinv2 * x_ref[...]
    w_ref[...] = (dinv * (s_ref[0] + s_ref[1]) + dinv2) * mask_ref[...]
    g1 = []
    for k in range(16):
        p = af * wg1_ref[k] + bg1_ref[k]
        g1.append(jnp.where(p > 0, p, 0.1 * p))
    for j in range(16):
        q = g1[0] * wg2_ref[0, j]
        for k in range(1, 16):
            q = q + g1[k] * wg2_ref[k, j]
        q_ref[j] = q
        qd_ref[j] = dinv * q


def _tc3_body(m_ref, q_ref, dinv_ref, w_ref, bg2_ref, t_ref):
    dinv = dinv_ref[...]
    dinv2 = dinv * dinv
    w = w_ref[...]
    for j in range(16):
        g2 = dinv * m_ref[j] + dinv2 * q_ref[j] + bg2_ref[j]
        g2 = jnp.where(g2 > 0, g2, 0.1 * g2)
        t_ref[0, j] = jnp.sum(w * g2)


def _tc4_body(meta_ref, t_ref, ws1_ref, bs1_ref, ws2_ref, bs2_ref, wso_ref,
              bso_ref, wg3_ref, bg3_ref, wf1_ref, bf1_ref, wf2_ref, bf2_ref,
              wfo_ref, bfo_ref, out_ref):
    def lrelu(a):
        return jnp.where(a > 0, a, 0.1 * a)

    def mm(a, b):
        return jnp.dot(a, b, preferred_element_type=jnp.float32)

    meta = jnp.broadcast_to(meta_ref[...], (8, 7))
    h = lrelu(mm(meta, ws1_ref[...]) + bs1_ref[...])
    h = lrelu(mm(h, ws2_ref[...]) + bs2_ref[...])
    out1 = mm(h, wso_ref[...]) + bso_ref[...]
    t = jnp.broadcast_to(t_ref[...], (8, 16))
    out2 = mm(t, wg3_ref[...]) * (1.0 / N_NODES) + bg3_ref[...]
    z = jnp.concatenate([out1, out2], axis=1)
    f = lrelu(mm(z, wf1_ref[...]) + bf1_ref[...])
    f = lrelu(mm(f, wf2_ref[...]) + bf2_ref[...])
    o = mm(f, wfo_ref[...]) + bfo_ref[...]
    out_ref[...] = 1.0 / (1.0 + jnp.exp(-o[0:1, :]))


def kernel(meta_vec, x, edge_index, Ws1, bs1, Ws2, bs2, Wso, bso,
           Wg1, bg1, Wg2, bg2, Wg3, bg3, Wf1, bf1, Wf2, bf2, Wfo, bfo):
    f32 = jnp.float32
    src32 = edge_index[0].astype(jnp.int32)
    dst32 = edge_index[1].astype(jnp.int32)

    def _stage(e, nshard, rows):
        epw = N_EDGES // nshard
        padw = rows * CH - epw
        pad = N_NODES + (jnp.arange(padw, dtype=jnp.int32) % NDUMMY)
        pad = jnp.broadcast_to(pad, (nshard, padw))
        e = e.reshape(nshard, epw)
        return jnp.concatenate([e, pad], axis=1).reshape(nshard, rows, CH)

    src = _stage(src32, NW, RPW)
    dst = _stage(dst32, NW, RPW)
    src_m0 = _stage(src32, NS, RPT)
    src_m = jnp.stack([src_m0, src_m0 + NPAD])
    dst_m = _stage(dst32, NS, RPT)

    xp = jnp.pad(x[:, 0], (0, NPAD - N_NODES)).reshape(R, L)
    mask = (jnp.arange(NPAD, dtype=jnp.int32) < N_NODES)
    mask = mask.astype(f32).reshape(R, L)

    deg = _deg_kernel(dst)

    dinv, v = pl.pallas_call(
        _tc1_body,
        out_shape=(jax.ShapeDtypeStruct((R, L), f32),
                   jax.ShapeDtypeStruct((R, L), f32)),
    )(deg.reshape(NC, R, L), xp)

    a_acc, s_acc = _as_kernel(src, dst, dinv.reshape(NPAD), v.reshape(NPAD))

    smem16 = pl.BlockSpec(memory_space=pltpu.SMEM)
    q, qd, w = pl.pallas_call(
        _tc2_body,
        in_specs=[pl.BlockSpec(), pl.BlockSpec(), pl.BlockSpec(),
                  pl.BlockSpec(), pl.BlockSpec(), smem16, smem16, smem16],
        out_shape=(jax.ShapeDtypeStruct((16, R, L), f32),
                   jax.ShapeDtypeStruct((16, R, L), f32),
                   jax.ShapeDtypeStruct((R, L), f32)),
    )(a_acc.reshape(NC, R, L), s_acc.reshape(NC, R, L), dinv, xp, mask,
      Wg1[0], bg1, Wg2)

    qd_rows = qd.reshape(16, NPAD).T
    qd_cat = jnp.concatenate([qd_rows[:, :8], qd_rows[:, 8:]], axis=0)
    zer8 = jnp.zeros((SLICE // 8, 8), f32)

    m_acc = _m_kernel(src_m, dst_m, qd_cat, zer8)

    m_planes = jnp.concatenate([m_acc[0], m_acc[1]], axis=1)
    m_planes = m_planes.T.reshape(16, R, L)

    t = pl.pallas_call(
        _tc3_body,
        in_specs=[pl.BlockSpec(), pl.BlockSpec(), pl.BlockSpec(),
                  pl.BlockSpec(), smem16],
        out_specs=pl.BlockSpec(memory_space=pltpu.SMEM),
        out_shape=jax.ShapeDtypeStruct((1, 16), f32),
    )(m_planes, q, dinv, w, bg2)

    out = pl.pallas_call(
        _tc4_body,
        out_shape=jax.ShapeDtypeStruct((1, 1), f32),
    )(meta_vec, t, Ws1, bs1.reshape(1, 64), Ws2, bs2.reshape(1, 64),
      Wso, bso.reshape(1, 16), Wg3, bg3.reshape(1, 16),
      Wf1, bf1.reshape(1, 64), Wf2, bf2.reshape(1, 32),
      Wfo, bfo.reshape(1, 1))

    return out.reshape(1)


# speedup vs baseline: 76.9273x; 1.2586x over previous
import functools

import jax
import jax.numpy as jnp
from jax import lax
from jax.experimental import pallas as pl
from jax.experimental.pallas import tpu as pltpu
from jax.experimental.pallas import tpu_sc as plsc

N_NODES = 100000
N_EDGES = 3200000
NPAD = 100352
R, L = 784, 128
NC, NS = 2, 16
NW = NC * NS
CH = 128
K = 8
EROWS = 25600
RPW = EROWS // NW
ROWS = 160
NBLK = RPW // ROWS
RPT = EROWS // NS
NBLK2 = RPT // ROWS
SLICE = NPAD // NS
NDUMMY = NPAD - N_NODES

_mesh = plsc.VectorSubcoreMesh(core_axis_name="c", subcore_axis_name="s")


def _fill(ref, n, value):
    val = jnp.full((16,), value, ref.dtype)

    def body(i, _):
        ref[pl.ds(i * 16, 16)] = val
        return 0

    lax.fori_loop(0, n // 16, body, 0)


@functools.partial(
    pl.kernel,
    out_type=jax.ShapeDtypeStruct((NC, NPAD), jnp.float32),
    mesh=_mesh,
    scratch_types=[
        pltpu.VMEM((ROWS, CH), jnp.int32),
        pltpu.VMEM((CH,), jnp.float32),
        pltpu.VMEM((SLICE,), jnp.float32),
        pltpu.VMEM_SHARED((NPAD,), jnp.float32),
        pltpu.SemaphoreType.DMA,
    ],
)
def _deg_kernel(dst_hbm, out_hbm, idx_v, ones_v, zer_v, acc_sh, ssem):
    c = lax.axis_index("c")
    s = lax.axis_index("s")
    wid = c * NS + s
    _fill(ones_v, CH, 1.0)
    _fill(zer_v, SLICE, 0.0)
    pltpu.sync_copy(zer_v, acc_sh.at[pl.ds(s * SLICE, SLICE)])
    plsc.subcore_barrier()

    for b in range(NBLK):
        pltpu.sync_copy(dst_hbm.at[pl.ds(wid * RPW + b * ROWS, ROWS)], idx_v)

        def step(jb, _):
            hs = [pltpu.async_copy(ones_v, acc_sh.at[idx_v.at[jb * K + i]],
                                   ssem, add=True) for i in range(K)]
            for h in hs:
                h.wait()
            return 0

        lax.fori_loop(0, ROWS // K, step, 0)

    plsc.subcore_barrier()
    pltpu.sync_copy(acc_sh.at[pl.ds(s * SLICE, SLICE)],
                    out_hbm.at[c, pl.ds(s * SLICE, SLICE)])


@functools.partial(
    pl.kernel,
    out_type=(jax.ShapeDtypeStruct((NC, NPAD), jnp.float32),
              jax.ShapeDtypeStruct((NC, NPAD), jnp.float32)),
    mesh=_mesh,
    scratch_types=[
        pltpu.VMEM((ROWS, CH), jnp.int32),
        pltpu.VMEM((ROWS, CH), jnp.int32),
        pltpu.VMEM((K, CH), jnp.float32),
        pltpu.VMEM((K, CH), jnp.float32),
        pltpu.VMEM((SLICE,), jnp.float32),
        pltpu.VMEM_SHARED((NPAD,), jnp.float32),
        pltpu.VMEM_SHARED((NPAD,), jnp.float32),
        pltpu.VMEM_SHARED((NPAD,), jnp.float32),
        pltpu.VMEM_SHARED((NPAD,), jnp.float32),
        pltpu.SemaphoreType.DMA,
        pltpu.SemaphoreType.DMA,
    ],
)
def _as_kernel(src_hbm, dst_hbm, u_hbm, v_hbm, a_out, s_out,
               sidx_v, didx_v, uva_v, vva_v, zer_v, u_sh, v_sh, a_sh, s_sh,
               gsem, ssem):
    c = lax.axis_index("c")
    s = lax.axis_index("s")
    wid = c * NS + s
    sl = pl.ds(s * SLICE, SLICE)
    _fill(zer_v, SLICE, 0.0)
    pltpu.sync_copy(zer_v, a_sh.at[sl])
    pltpu.sync_copy(zer_v, s_sh.at[sl])
    pltpu.sync_copy(u_hbm.at[sl], u_sh.at[sl])
    pltpu.sync_copy(v_hbm.at[sl], v_sh.at[sl])
    plsc.subcore_barrier()

    for b in range(NBLK):
        pltpu.sync_copy(src_hbm.at[pl.ds(wid * RPW + b * ROWS, ROWS)], sidx_v)
        pltpu.sync_copy(dst_hbm.at[pl.ds(wid * RPW + b * ROWS, ROWS)], didx_v)

        def step(jb, _):
            gs = []
            for i in range(K):
                j = jb * K + i
                gs.append(pltpu.async_copy(v_sh.at[sidx_v.at[j]],
                                           vva_v.at[i], gsem))
                gs.append(pltpu.async_copy(u_sh.at[didx_v.at[j]],
                                           uva_v.at[i], gsem))
            for h in gs:
                h.wait()
            ss = []
            for i in range(K):
                j = jb * K + i
                ss.append(pltpu.async_copy(vva_v.at[i],
                                           a_sh.at[didx_v.at[j]], ssem,
                                           add=True))
                ss.append(pltpu.async_copy(uva_v.at[i],
                                           s_sh.at[sidx_v.at[j]], ssem,
                                           add=True))
            for h in ss:
                h.wait()
            return 0

        lax.fori_loop(0, ROWS // K, step, 0)

    plsc.subcore_barrier()
    pltpu.sync_copy(a_sh.at[sl], a_out.at[c, sl])
    pltpu.sync_copy(s_sh.at[sl], s_out.at[c, sl])


@functools.partial(
    pl.kernel,
    out_type=jax.ShapeDtypeStruct((NC, NPAD, 8), jnp.float32),
    mesh=_mesh,
    compiler_params=pltpu.CompilerParams(use_tc_tiling_on_sc=False),
    scratch_types=[
        pltpu.VMEM((ROWS, CH), jnp.int32),
        pltpu.VMEM((ROWS, CH), jnp.int32),
        pltpu.VMEM((K, CH, 8), jnp.float32),
        pltpu.VMEM_SHARED((NPAD, 8), jnp.float32),
        pltpu.SemaphoreType.DMA,
        pltpu.SemaphoreType.DMA,
    ],
)
def _m_kernel(src_hbm, dst_hbm, qd_lo, qd_hi, zer_hbm, out_hbm,
              sidx_v, didx_v, rows_v, m_sh, gsem, ssem):
    c = lax.axis_index("c")
    s = lax.axis_index("s")
    for kk in range(8):
        pltpu.sync_copy(zer_hbm,
                        m_sh.at[pl.ds(s * SLICE + kk * (SLICE // 8),
                                      SLICE // 8)])
    plsc.subcore_barrier()

    def run(qd_hbm):
        for b in range(NBLK2):
            pltpu.sync_copy(src_hbm.at[pl.ds(s * RPT + b * ROWS, ROWS)],
                            sidx_v)
            pltpu.sync_copy(dst_hbm.at[pl.ds(s * RPT + b * ROWS, ROWS)],
                            didx_v)

            def step(jb, _):
                gs = [pltpu.async_copy(qd_hbm.at[sidx_v.at[jb * K + i]],
                                       rows_v.at[i], gsem) for i in range(K)]
                for h in gs:
                    h.wait()
                ss = [pltpu.async_copy(rows_v.at[i],
                                       m_sh.at[didx_v.at[jb * K + i]], ssem,
                                       add=True) for i in range(K)]
                for h in ss:
                    h.wait()
                return 0

            lax.fori_loop(0, ROWS // K, step, 0)

    @pl.when(c == 0)
    def _():
        run(qd_lo)

    @pl.when(c == 1)
    def _():
        run(qd_hi)

    plsc.subcore_barrier()
    sl = pl.ds(s * SLICE, SLICE)
    pltpu.sync_copy(m_sh.at[sl], out_hbm.at[c, sl])


def _tc1_body(deg_ref, x_ref, dinv_ref, v_ref):
    deg = deg_ref[0] + deg_ref[1] + 1.0
    dinv = lax.rsqrt(deg)
    dinv_ref[...] = dinv
    v_ref[...] = dinv * x_ref[...]


def _tc2_body(a_ref, s_ref, dinv_ref, x_ref, mask_ref, wg1_ref, bg1_ref,
              wg2_ref, q_ref, qd_ref, w_ref):
    dinv = dinv_ref[...]
    dinv2 = dinv * dinv
    af = dinv * (a_ref[0] + a_ref[1]) + dinv2 * x_ref[...]
    w_ref[...] = (dinv * (s_ref[0] + s_ref[1]) + dinv2) * mask_ref[...]
    g1 = []
    for k in range(16):
        p = af * wg1_ref[k] + bg1_ref[k]
        g1.append(jnp.where(p > 0, p, 0.1 * p))
    for j in range(16):
        q = g1[0] * wg2_ref[0, j]
        for k in range(1, 16):
            q = q + g1[k] * wg2_ref[k, j]
        q_ref[j] = q
        qd_ref[j] = dinv * q


def _tc3_body(m_ref, q_ref, dinv_ref, w_ref, bg2_ref, t_ref):
    dinv = dinv_ref[...]
    dinv2 = dinv * dinv
    w = w_ref[...]
    for j in range(16):
        g2 = dinv * m_ref[j] + dinv2 * q_ref[j] + bg2_ref[j]
        g2 = jnp.where(g2 > 0, g2, 0.1 * g2)
        t_ref[0, j] = jnp.sum(w * g2)


def _tc4_body(meta_ref, t_ref, ws1_ref, bs1_ref, ws2_ref, bs2_ref, wso_ref,
              bso_ref, wg3_ref, bg3_ref, wf1_ref, bf1_ref, wf2_ref, bf2_ref,
              wfo_ref, bfo_ref, out_ref):
    def lrelu(a):
        return jnp.where(a > 0, a, 0.1 * a)

    def mm(a, b):
        return jnp.dot(a, b, preferred_element_type=jnp.float32)

    meta = jnp.broadcast_to(meta_ref[...], (8, 7))
    h = lrelu(mm(meta, ws1_ref[...]) + bs1_ref[...])
    h = lrelu(mm(h, ws2_ref[...]) + bs2_ref[...])
    out1 = mm(h, wso_ref[...]) + bso_ref[...]
    t = jnp.broadcast_to(t_ref[...], (8, 16))
    out2 = mm(t, wg3_ref[...]) * (1.0 / N_NODES) + bg3_ref[...]
    z = jnp.concatenate([out1, out2], axis=1)
    f = lrelu(mm(z, wf1_ref[...]) + bf1_ref[...])
    f = lrelu(mm(f, wf2_ref[...]) + bf2_ref[...])
    o = mm(f, wfo_ref[...]) + bfo_ref[...]
    out_ref[...] = 1.0 / (1.0 + jnp.exp(-o[0:1, :]))


def kernel(meta_vec, x, edge_index, Ws1, bs1, Ws2, bs2, Wso, bso,
           Wg1, bg1, Wg2, bg2, Wg3, bg3, Wf1, bf1, Wf2, bf2, Wfo, bfo):
    f32 = jnp.float32
    npadedge = EROWS * CH - N_EDGES
    padidx = N_NODES + (jnp.arange(npadedge, dtype=jnp.int32) % NDUMMY)

    def _stage(e):
        e = e.astype(jnp.int32)
        return jnp.concatenate([e, padidx]).reshape(EROWS, CH)

    src = _stage(edge_index[0])
    dst = _stage(edge_index[1])

    xp = jnp.pad(x[:, 0], (0, NPAD - N_NODES)).reshape(R, L)
    mask = (jnp.arange(NPAD, dtype=jnp.int32) < N_NODES)
    mask = mask.astype(f32).reshape(R, L)

    deg = _deg_kernel(dst)

    dinv, v = pl.pallas_call(
        _tc1_body,
        out_shape=(jax.ShapeDtypeStruct((R, L), f32),
                   jax.ShapeDtypeStruct((R, L), f32)),
    )(deg.reshape(NC, R, L), xp)

    a_acc, s_acc = _as_kernel(src, dst, dinv.reshape(NPAD), v.reshape(NPAD))

    smem16 = pl.BlockSpec(memory_space=pltpu.SMEM)
    q, qd, w = pl.pallas_call(
        _tc2_body,
        in_specs=[pl.BlockSpec(), pl.BlockSpec(), pl.BlockSpec(),
                  pl.BlockSpec(), pl.BlockSpec(), smem16, smem16, smem16],
        out_shape=(jax.ShapeDtypeStruct((16, R, L), f32),
                   jax.ShapeDtypeStruct((16, R, L), f32),
                   jax.ShapeDtypeStruct((R, L), f32)),
    )(a_acc.reshape(NC, R, L), s_acc.reshape(NC, R, L), dinv, xp, mask,
      Wg1[0], bg1, Wg2)

    qd_rows = qd.reshape(16, NPAD).T
    zer8 = jnp.zeros((SLICE // 8, 8), f32)

    m_acc = _m_kernel(src, dst, qd_rows[:, :8], qd_rows[:, 8:], zer8)

    m_planes = jnp.concatenate([m_acc[0], m_acc[1]], axis=1)
    m_planes = m_planes.T.reshape(16, R, L)

    t = pl.pallas_call(
        _tc3_body,
        in_specs=[pl.BlockSpec(), pl.BlockSpec(), pl.BlockSpec(),
                  pl.BlockSpec(), smem16],
        out_specs=pl.BlockSpec(memory_space=pltpu.SMEM),
        out_shape=jax.ShapeDtypeStruct((1, 16), f32),
    )(m_planes, q, dinv, w, bg2)

    out = pl.pallas_call(
        _tc4_body,
        out_shape=jax.ShapeDtypeStruct((1, 1), f32),
    )(meta_vec, t, Ws1, bs1.reshape(1, 64), Ws2, bs2.reshape(1, 64),
      Wso, bso.reshape(1, 16), Wg3, bg3.reshape(1, 16),
      Wf1, bf1.reshape(1, 64), Wf2, bf2.reshape(1, 32),
      Wfo, bfo.reshape(1, 1))

    return out.reshape(1)


# speedup vs baseline: 78.5789x; 1.0215x over previous
import functools

import jax
import jax.numpy as jnp
from jax import lax
from jax.experimental import pallas as pl
from jax.experimental.pallas import tpu as pltpu
from jax.experimental.pallas import tpu_sc as plsc

N_NODES = 100000
N_EDGES = 3200000
NPAD = 100352
R, L = 784, 128
NC, NS = 2, 16
NW = NC * NS
CH = 128
K = 8
K2 = 8
EROWS = 25600
RPW = EROWS // NW
ROWS = 160
NBLK = RPW // ROWS
RPT = EROWS // NS
NBLK2 = RPT // ROWS
SLICE = NPAD // NS
NDUMMY = NPAD - N_NODES

_mesh = plsc.VectorSubcoreMesh(core_axis_name="c", subcore_axis_name="s")


def _fill(ref, n, value):
    val = jnp.full((16,), value, ref.dtype)

    def body(i, _):
        ref[pl.ds(i * 16, 16)] = val
        return 0

    lax.fori_loop(0, n // 16, body, 0)


@functools.partial(
    pl.kernel,
    out_type=jax.ShapeDtypeStruct((NC, NPAD), jnp.float32),
    mesh=_mesh,
    scratch_types=[
        pltpu.VMEM((ROWS, CH), jnp.int32),
        pltpu.VMEM((CH,), jnp.float32),
        pltpu.VMEM((SLICE,), jnp.float32),
        pltpu.VMEM_SHARED((NPAD,), jnp.float32),
        pltpu.SemaphoreType.DMA,
    ],
)
def _deg_kernel(dst_hbm, out_hbm, idx_v, ones_v, zer_v, acc_sh, ssem):
    c = lax.axis_index("c")
    s = lax.axis_index("s")
    wid = c * NS + s
    _fill(ones_v, CH, 1.0)
    _fill(zer_v, SLICE, 0.0)
    pltpu.sync_copy(zer_v, acc_sh.at[pl.ds(s * SLICE, SLICE)])
    plsc.subcore_barrier()

    for b in range(NBLK):
        pltpu.sync_copy(dst_hbm.at[pl.ds(wid * RPW + b * ROWS, ROWS)], idx_v)

        def step(jb, _):
            hs = [pltpu.async_copy(ones_v, acc_sh.at[idx_v.at[jb * K + i]],
                                   ssem, add=True) for i in range(K)]
            for h in hs:
                h.wait()
            return 0

        lax.fori_loop(0, ROWS // K, step, 0)

    plsc.subcore_barrier()
    pltpu.sync_copy(acc_sh.at[pl.ds(s * SLICE, SLICE)],
                    out_hbm.at[c, pl.ds(s * SLICE, SLICE)])


@functools.partial(
    pl.kernel,
    out_type=(jax.ShapeDtypeStruct((NC, NPAD), jnp.float32),
              jax.ShapeDtypeStruct((NC, NPAD), jnp.float32)),
    mesh=_mesh,
    scratch_types=[
        pltpu.VMEM((ROWS, CH), jnp.int32),
        pltpu.VMEM((ROWS, CH), jnp.int32),
        pltpu.VMEM((K, CH), jnp.float32),
        pltpu.VMEM((K, CH), jnp.float32),
        pltpu.VMEM((SLICE,), jnp.float32),
        pltpu.VMEM_SHARED((NPAD,), jnp.float32),
        pltpu.VMEM_SHARED((NPAD,), jnp.float32),
        pltpu.VMEM_SHARED((NPAD,), jnp.float32),
        pltpu.VMEM_SHARED((NPAD,), jnp.float32),
        pltpu.SemaphoreType.DMA,
        pltpu.SemaphoreType.DMA,
    ],
)
def _as_kernel(src_hbm, dst_hbm, u_hbm, v_hbm, a_out, s_out,
               sidx_v, didx_v, uva_v, vva_v, zer_v, u_sh, v_sh, a_sh, s_sh,
               gsem, ssem):
    c = lax.axis_index("c")
    s = lax.axis_index("s")
    wid = c * NS + s
    sl = pl.ds(s * SLICE, SLICE)
    _fill(zer_v, SLICE, 0.0)
    pltpu.sync_copy(zer_v, a_sh.at[sl])
    pltpu.sync_copy(zer_v, s_sh.at[sl])
    pltpu.sync_copy(u_hbm.at[sl], u_sh.at[sl])
    pltpu.sync_copy(v_hbm.at[sl], v_sh.at[sl])
    plsc.subcore_barrier()

    for b in range(NBLK):
        pltpu.sync_copy(src_hbm.at[pl.ds(wid * RPW + b * ROWS, ROWS)], sidx_v)
        pltpu.sync_copy(dst_hbm.at[pl.ds(wid * RPW + b * ROWS, ROWS)], didx_v)

        def step(jb, _):
            gs = []
            for i in range(K):
                j = jb * K + i
                gs.append(pltpu.async_copy(v_sh.at[sidx_v.at[j]],
                                           vva_v.at[i], gsem))
                gs.append(pltpu.async_copy(u_sh.at[didx_v.at[j]],
                                           uva_v.at[i], gsem))
            for h in gs:
                h.wait()
            ss = []
            for i in range(K):
                j = jb * K + i
                ss.append(pltpu.async_copy(vva_v.at[i],
                                           a_sh.at[didx_v.at[j]], ssem,
                                           add=True))
                ss.append(pltpu.async_copy(uva_v.at[i],
                                           s_sh.at[sidx_v.at[j]], ssem,
                                           add=True))
            for h in ss:
                h.wait()
            return 0

        lax.fori_loop(0, ROWS // K, step, 0)

    plsc.subcore_barrier()
    pltpu.sync_copy(a_sh.at[sl], a_out.at[c, sl])
    pltpu.sync_copy(s_sh.at[sl], s_out.at[c, sl])


@functools.partial(
    pl.kernel,
    out_type=jax.ShapeDtypeStruct((NC, NPAD, 8), jnp.float32),
    mesh=_mesh,
    compiler_params=pltpu.CompilerParams(use_tc_tiling_on_sc=False),
    scratch_types=[
        pltpu.VMEM((ROWS, CH), jnp.int32),
        pltpu.VMEM((ROWS, CH), jnp.int32),
        pltpu.VMEM((K2, CH, 8), jnp.float32),
        pltpu.VMEM_SHARED((NPAD, 8), jnp.float32),
        pltpu.SemaphoreType.DMA,
        pltpu.SemaphoreType.DMA,
    ],
)
def _m_kernel(src_hbm, dst_hbm, qd_lo, qd_hi, zer_hbm, out_hbm,
              sidx_v, didx_v, rows_v, m_sh, gsem, ssem):
    c = lax.axis_index("c")
    s = lax.axis_index("s")
    for kk in range(8):
        pltpu.sync_copy(zer_hbm,
                        m_sh.at[pl.ds(s * SLICE + kk * (SLICE // 8),
                                      SLICE // 8)])
    plsc.subcore_barrier()

    def run(qd_hbm):
        for b in range(NBLK2):
            pltpu.sync_copy(src_hbm.at[pl.ds(s * RPT + b * ROWS, ROWS)],
                            sidx_v)
            pltpu.sync_copy(dst_hbm.at[pl.ds(s * RPT + b * ROWS, ROWS)],
                            didx_v)

            def step(jb, _):
                gs = [pltpu.async_copy(qd_hbm.at[sidx_v.at[jb * K2 + i]],
                                       rows_v.at[i], gsem) for i in range(K2)]
                for h in gs:
                    h.wait()
                ss = [pltpu.async_copy(rows_v.at[i],
                                       m_sh.at[didx_v.at[jb * K2 + i]], ssem,
                                       add=True) for i in range(K2)]
                for h in ss:
                    h.wait()
                return 0

            lax.fori_loop(0, ROWS // K2, step, 0)

    @pl.when(c == 0)
    def _():
        run(qd_lo)

    @pl.when(c == 1)
    def _():
        run(qd_hi)

    plsc.subcore_barrier()
    sl = pl.ds(s * SLICE, SLICE)
    pltpu.sync_copy(m_sh.at[sl], out_hbm.at[c, sl])


def _tc1_body(deg_ref, x_ref, dinv_ref, v_ref):
    deg = deg_ref[0] + deg_ref[1] + 1.0
    dinv = lax.rsqrt(deg)
    dinv_ref[...] = dinv
    v_ref[...] = dinv * x_ref[...]


def _tc2_body(a_ref, s_ref, dinv_ref, x_ref, mask_ref, wg1_ref, bg1_ref,
              wg2_ref, q_ref, qd_ref, w_ref):
    dinv = dinv_ref[...]
    dinv2 = dinv * dinv
    af = dinv * (a_ref[0] + a_ref[1]) + dinv2 * x_ref[...]
    w_ref[...] = (dinv * (s_ref[0] + s_ref[1]) + dinv2) * mask_ref[...]
    g1 = []
    for k in range(16):
        p = af * wg1_ref[k] + bg1_ref[k]
        g1.append(jnp.where(p > 0, p, 0.1 * p))
    for j in range(16):
        q = g1[0] * wg2_ref[0, j]
        for k in range(1, 16):
            q = q + g1[k] * wg2_ref[k, j]
        q_ref[j] = q
        qd_ref[j] = dinv * q


def _tc3_body(m_ref, q_ref, dinv_ref, w_ref, bg2_ref, t_ref):
    dinv = dinv_ref[...]
    dinv2 = dinv * dinv
    w = w_ref[...]
    for j in range(16):
        g2 = dinv * m_ref[j] + dinv2 * q_ref[j] + bg2_ref[j]
        g2 = jnp.where(g2 > 0, g2, 0.1 * g2)
        t_ref[0, j] = jnp.sum(w * g2)


def _tc4_body(meta_ref, t_ref, ws1_ref, bs1_ref, ws2_ref, bs2_ref, wso_ref,
              bso_ref, wg3_ref, bg3_ref, wf1_ref, bf1_ref, wf2_ref, bf2_ref,
              wfo_ref, bfo_ref, out_ref):
    def lrelu(a):
        return jnp.where(a > 0, a, 0.1 * a)

    def mm(a, b):
        return jnp.dot(a, b, preferred_element_type=jnp.float32)

    meta = jnp.broadcast_to(meta_ref[...], (8, 7))
    h = lrelu(mm(meta, ws1_ref[...]) + bs1_ref[...])
    h = lrelu(mm(h, ws2_ref[...]) + bs2_ref[...])
    out1 = mm(h, wso_ref[...]) + bso_ref[...]
    t = jnp.broadcast_to(t_ref[...], (8, 16))
    out2 = mm(t, wg3_ref[...]) * (1.0 / N_NODES) + bg3_ref[...]
    z = jnp.concatenate([out1, out2], axis=1)
    f = lrelu(mm(z, wf1_ref[...]) + bf1_ref[...])
    f = lrelu(mm(f, wf2_ref[...]) + bf2_ref[...])
    o = mm(f, wfo_ref[...]) + bfo_ref[...]
    out_ref[...] = 1.0 / (1.0 + jnp.exp(-o[0:1, :]))


def kernel(meta_vec, x, edge_index, Ws1, bs1, Ws2, bs2, Wso, bso,
           Wg1, bg1, Wg2, bg2, Wg3, bg3, Wf1, bf1, Wf2, bf2, Wfo, bfo):
    f32 = jnp.float32
    npadedge = EROWS * CH - N_EDGES
    padidx = N_NODES + (jnp.arange(npadedge, dtype=jnp.int32) % NDUMMY)

    def _stage(e):
        e = e.astype(jnp.int32)
        return jnp.concatenate([e, padidx]).reshape(EROWS, CH)

    src = _stage(edge_index[0])
    dst = _stage(edge_index[1])

    xp = jnp.pad(x[:, 0], (0, NPAD - N_NODES)).reshape(R, L)
    mask = (jnp.arange(NPAD, dtype=jnp.int32) < N_NODES)
    mask = mask.astype(f32).reshape(R, L)

    deg = _deg_kernel(dst)

    dinv, v = pl.pallas_call(
        _tc1_body,
        out_shape=(jax.ShapeDtypeStruct((R, L), f32),
                   jax.ShapeDtypeStruct((R, L), f32)),
    )(deg.reshape(NC, R, L), xp)

    a_acc, s_acc = _as_kernel(src, dst, dinv.reshape(NPAD), v.reshape(NPAD))

    smem16 = pl.BlockSpec(memory_space=pltpu.SMEM)
    q, qd, w = pl.pallas_call(
        _tc2_body,
        in_specs=[pl.BlockSpec(), pl.BlockSpec(), pl.BlockSpec(),
                  pl.BlockSpec(), pl.BlockSpec(), smem16, smem16, smem16],
        out_shape=(jax.ShapeDtypeStruct((16, R, L), f32),
                   jax.ShapeDtypeStruct((16, R, L), f32),
                   jax.ShapeDtypeStruct((R, L), f32)),
    )(a_acc.reshape(NC, R, L), s_acc.reshape(NC, R, L), dinv, xp, mask,
      Wg1[0], bg1, Wg2)

    qd_p = qd.reshape(2, 8, NPAD // 16, 16).transpose(0, 2, 3, 1)
    qd_p = qd_p.reshape(2, NPAD, 8)
    zer8 = jnp.zeros((SLICE // 8, 8), f32)

    m_acc = _m_kernel(src, dst, qd_p[0], qd_p[1], zer8)

    m_planes = m_acc.reshape(NC, NPAD // 16, 16, 8).transpose(0, 3, 1, 2)
    m_planes = m_planes.reshape(16, R, L)

    t = pl.pallas_call(
        _tc3_body,
        in_specs=[pl.BlockSpec(), pl.BlockSpec(), pl.BlockSpec(),
                  pl.BlockSpec(), smem16],
        out_specs=pl.BlockSpec(memory_space=pltpu.SMEM),
        out_shape=jax.ShapeDtypeStruct((1, 16), f32),
    )(m_planes, q, dinv, w, bg2)

    out = pl.pallas_call(
        _tc4_body,
        out_shape=jax.ShapeDtypeStruct((1, 1), f32),
    )(meta_vec, t, Ws1, bs1.reshape(1, 64), Ws2, bs2.reshape(1, 64),
      Wso, bso.reshape(1, 16), Wg3, bg3.reshape(1, 16),
      Wf1, bf1.reshape(1, 64), Wf2, bf2.reshape(1, 32),
      Wfo, bfo.reshape(1, 1))

    return out.reshape(1)
